# baseline (speedup 1.0000x reference)
import functools

import jax
import jax.numpy as jnp
from jax import lax
from jax.experimental import pallas as pl
from jax.experimental.pallas import tpu as pltpu
from jax.experimental.pallas import tpu_sc as plsc

_D = 128
_IW = 16
_CHUNK = 128
_NW = 32
_NB = 4
_NI = 2
_L = 16


def _gather_body(nchunks, tail, x_hbm, inds_hbm, out_hbm, *refs):
  iblk = refs[0:_NI]
  idxv = refs[_NI:_NI + _NB]
  rows = refs[_NI + _NB:_NI + 2 * _NB]
  isem = refs[_NI + 2 * _NB:2 * _NI + 2 * _NB]
  gsem = refs[2 * _NI + 2 * _NB:2 * _NI + 3 * _NB]
  ssem = refs[2 * _NI + 3 * _NB:2 * _NI + 4 * _NB]

  big = (nchunks + _NW - 1) // _NW
  cut = nchunks - (big - 1) * _NW

  w = lax.axis_index("s") * 2 + lax.axis_index("c")
  nj = jnp.where(w < cut, big, big - 1)
  base_chunk = jnp.where(w < cut, w * big, cut * big + (w - cut) * (big - 1))

  def start_iblk(k, q):
    gc = base_chunk + k
    if tail == _CHUNK:
      pltpu.async_copy(inds_hbm.at[pl.ds(gc * _CHUNK, _CHUNK)], iblk[q],
                       isem[q])
    else:
      @pl.when(gc != nchunks - 1)
      def _f():
        pltpu.async_copy(inds_hbm.at[pl.ds(gc * _CHUNK, _CHUNK)], iblk[q],
                         isem[q])

      @pl.when(gc == nchunks - 1)
      def _t():
        pltpu.async_copy(inds_hbm.at[pl.ds(gc * _CHUNK, tail)],
                         iblk[q].at[pl.ds(0, tail)], isem[q])

  def extract(k, q, b):
    gc = base_chunk + k
    zeros = jnp.zeros((_L,), jnp.int32)

    def wait_i(n):
      pltpu.make_async_copy(inds_hbm.at[pl.ds(0, n)],
                            iblk[q].at[pl.ds(0, n)], isem[q]).wait()

    def pick(g):
      r = lax.iota(jnp.int32, _L) + g * _L
      idxv[b][pl.ds(g * _L, _L)] = plsc.load_gather(iblk[q], [r, zeros])

    if tail == _CHUNK:
      wait_i(_CHUNK)
      for g in range(_CHUNK // _L):
        pick(g)
    else:
      @pl.when(gc != nchunks - 1)
      def _f():
        wait_i(_CHUNK)
        for g in range(_CHUNK // _L):
          pick(g)

      @pl.when(gc == nchunks - 1)
      def _t():
        wait_i(tail)
        for g in range(tail // _L):
          pick(g)
        for g in range(tail // _L, _CHUNK // _L):
          idxv[b][pl.ds(g * _L, _L)] = zeros

  def start_gather(b):
    pltpu.async_copy(x_hbm.at[idxv[b]], rows[b], gsem[b])

  def wait_gather(b):
    pltpu.make_async_copy(x_hbm.at[idxv[b]], rows[b], gsem[b]).wait()

  def start_store(j, b):
    gc = base_chunk + j
    if tail == _CHUNK:
      pltpu.async_copy(rows[b], out_hbm.at[pl.ds(gc * _CHUNK, _CHUNK)], ssem[b])
    else:
      @pl.when(gc != nchunks - 1)
      def _f():
        pltpu.async_copy(rows[b], out_hbm.at[pl.ds(gc * _CHUNK, _CHUNK)],
                         ssem[b])

      @pl.when(gc == nchunks - 1)
      def _t():
        pltpu.async_copy(rows[b].at[pl.ds(0, tail)],
                         out_hbm.at[pl.ds(gc * _CHUNK, tail)], ssem[b])

  def wait_store(b, is_tail):
    n = tail if is_tail else _CHUNK
    pltpu.make_async_copy(rows[b].at[pl.ds(0, n)],
                          out_hbm.at[pl.ds(0, n)], ssem[b]).wait()

  @pl.when(0 < nj)
  def _p0():
    start_iblk(0, 0)

  @pl.when(1 < nj)
  def _p1():
    start_iblk(1, 1)

  @pl.when(0 < nj)
  def _p2():
    extract(0, 0, 0)
    start_gather(0)

  @pl.when(2 < nj)
  def _p3():
    start_iblk(2, 0)

  @pl.when(1 < nj)
  def _p4():
    extract(1, 1, 1)
    start_gather(1)

  nrounds = (nj + _NB - 1) // _NB

  @pl.loop(0, nrounds)
  def _round(r):
    for p in range(_NB):
      j = r * _NB + p

      @pl.when(j < nj)
      def _body(j=j, p=p):
        pm1 = (p - 1) % _NB

        @pl.when(j >= 1)
        def _drain_prev():
          wait_store(pm1, False)

        @pl.when(j + 3 < nj)
        def _prefetch_i():
          start_iblk(j + 3, (p + 1) % _NI)

        @pl.when(j + 2 < nj)
        def _launch_g():
          extract(j + 2, p % _NI, (p + 2) % _NB)
          start_gather((p + 2) % _NB)

        wait_gather(p)
        start_store(j, p)

  last_p = lax.rem(nj - 1, _NB)
  last_is_tail = (base_chunk + nj - 1) == (nchunks - 1)
  for p in range(_NB):
    @pl.when(last_p == p)
    def _drain_last(p=p):
      if tail == _CHUNK:
        wait_store(p, False)
      else:
        @pl.when(last_is_tail)
        def _t():
          wait_store(p, True)

        @pl.when(jnp.logical_not(last_is_tail))
        def _f():
          wait_store(p, False)


@functools.partial(jax.jit, static_argnums=(2, 3))
def _gather(x, inds, nchunks, tail):
  n_out = inds.shape[0]
  mesh = plsc.VectorSubcoreMesh(core_axis_name="c", subcore_axis_name="s")
  run = pl.kernel(
      functools.partial(_gather_body, nchunks, tail),
      out_type=jax.ShapeDtypeStruct((n_out, _D), jnp.float32),
      mesh=mesh,
      scratch_types=[pltpu.VMEM((_CHUNK, _IW), jnp.int32) for _ in range(_NI)]
      + [pltpu.VMEM((_CHUNK,), jnp.int32) for _ in range(_NB)]
      + [pltpu.VMEM((_CHUNK, _D), jnp.float32) for _ in range(_NB)]
      + [pltpu.SemaphoreType.DMA for _ in range(_NI + 2 * _NB)],
      compiler_params=pltpu.CompilerParams(needs_layout_passes=False),
  )
  return run(x, inds)


def kernel(x, upsample_inds):
  n_out = upsample_inds.shape[0]
  inds = upsample_inds.astype(jnp.int32)
  nchunks = (n_out + _CHUNK - 1) // _CHUNK
  tail = n_out - (nchunks - 1) * _CHUNK
  return _gather(x, inds, nchunks, tail)


# speedup vs baseline: 1.7509x; 1.7509x over previous
import functools

import jax
import jax.numpy as jnp
from jax import lax
from jax.experimental import pallas as pl
from jax.experimental.pallas import tpu as pltpu
from jax.experimental.pallas import tpu_sc as plsc

_D = 128
_CHUNK = 128
_NW = 32
_NB = 6


def _gather_body(nchunks, tail, x_hbm, idx_hbm, out_hbm, *refs):
  ibuf = refs[0]
  rows = refs[1:1 + _NB]
  gsem = refs[1 + _NB:1 + 2 * _NB]
  hsem = refs[1 + 2 * _NB:1 + 3 * _NB]
  ssem = refs[1 + 3 * _NB:1 + 4 * _NB]

  big = (nchunks + _NW - 1) // _NW
  cut = nchunks - (big - 1) * _NW

  w = lax.axis_index("s") * 2 + lax.axis_index("c")
  nj = jnp.where(w < cut, big, big - 1)
  base_chunk = jnp.where(w < cut, w * big, cut * big + (w - cut) * (big - 1))

  pltpu.sync_copy(idx_hbm.at[pl.ds(base_chunk * _CHUNK, big * _CHUNK)], ibuf)

  _H = _CHUNK // 2

  def start_gather(j, p):
    pltpu.async_copy(
        x_hbm.at[ibuf.at[pl.ds(j * _CHUNK, _H)]],
        rows[p].at[pl.ds(0, _H)], gsem[p])
    pltpu.async_copy(
        x_hbm.at[ibuf.at[pl.ds(j * _CHUNK + _H, _H)]],
        rows[p].at[pl.ds(_H, _H)], hsem[p])

  def wait_gather(p):
    pltpu.make_async_copy(
        x_hbm.at[ibuf.at[pl.ds(0, _H)]],
        rows[p].at[pl.ds(0, _H)], gsem[p]).wait()
    pltpu.make_async_copy(
        x_hbm.at[ibuf.at[pl.ds(0, _H)]],
        rows[p].at[pl.ds(_H, _H)], hsem[p]).wait()

  def start_store(j, p):
    gc = base_chunk + j
    if tail == _CHUNK:
      pltpu.async_copy(rows[p], out_hbm.at[pl.ds(gc * _CHUNK, _CHUNK)], ssem[p])
    else:
      @pl.when(gc == nchunks - 1)
      def _t():
        pltpu.async_copy(rows[p].at[pl.ds(0, tail)],
                         out_hbm.at[pl.ds(gc * _CHUNK, tail)], ssem[p])

      @pl.when(gc != nchunks - 1)
      def _f():
        pltpu.async_copy(rows[p], out_hbm.at[pl.ds(gc * _CHUNK, _CHUNK)],
                         ssem[p])

  def wait_store(p, is_tail):
    n = tail if is_tail else _CHUNK
    pltpu.make_async_copy(rows[p].at[pl.ds(0, n)],
                          out_hbm.at[pl.ds(0, n)], ssem[p]).wait()

  for p in range(_NB - 1):
    @pl.when(p < nj)
    def _prime(p=p):
      start_gather(p, p)

  nrounds = (nj + _NB - 1) // _NB

  @pl.loop(0, nrounds)
  def _round(r):
    for p in range(_NB):
      j = r * _NB + p

      @pl.when(j < nj)
      def _body(j=j, p=p):
        pm1 = (p - 1) % _NB

        @pl.when(j >= 1)
        def _drain_prev():
          wait_store(pm1, False)

        @pl.when(j + _NB - 1 < nj)
        def _prefetch():
          start_gather(j + _NB - 1, pm1)

        wait_gather(p)
        start_store(j, p)

  last_p = lax.rem(nj - 1, _NB)
  last_is_tail = (base_chunk + nj - 1) == (nchunks - 1)
  for p in range(_NB):
    @pl.when(last_p == p)
    def _drain_last(p=p):
      if tail == _CHUNK:
        wait_store(p, False)
      else:
        @pl.when(last_is_tail)
        def _t():
          wait_store(p, True)

        @pl.when(jnp.logical_not(last_is_tail))
        def _f():
          wait_store(p, False)


@functools.partial(jax.jit, static_argnums=(2, 3, 4))
def _gather(x, idx_pad, n_out, nchunks, tail):
  big = (nchunks + _NW - 1) // _NW
  mesh = plsc.VectorSubcoreMesh(core_axis_name="c", subcore_axis_name="s")
  run = pl.kernel(
      functools.partial(_gather_body, nchunks, tail),
      out_type=jax.ShapeDtypeStruct((n_out, _D), jnp.float32),
      mesh=mesh,
      scratch_types=[pltpu.VMEM((big * _CHUNK,), jnp.int32)]
      + [pltpu.VMEM((_CHUNK, _D), jnp.float32) for _ in range(_NB)]
      + [pltpu.SemaphoreType.DMA for _ in range(3 * _NB)],
  )
  return run(x, idx_pad)


def kernel(x, upsample_inds):
  n_out = upsample_inds.shape[0]
  idx = upsample_inds[:, 0].astype(jnp.int32)
  nchunks = (n_out + _CHUNK - 1) // _CHUNK
  tail = n_out - (nchunks - 1) * _CHUNK
  big = (nchunks + _NW - 1) // _NW
  cut = nchunks - (big - 1) * _NW
  last_base = cut * big + (_NW - 1 - cut) * (big - 1)
  pad_len = (last_base + big) * _CHUNK
  idx_pad = jnp.pad(idx, (0, pad_len - n_out))
  return _gather(x, idx_pad, n_out, nchunks, tail)


# speedup vs baseline: 1.7577x; 1.0039x over previous
import functools

import jax
import jax.numpy as jnp
from jax import lax
from jax.experimental import pallas as pl
from jax.experimental.pallas import tpu as pltpu
from jax.experimental.pallas import tpu_sc as plsc

_D = 128
_CHUNK = 128
_NW = 32
_NB = 6


def _gather_body(nchunks, tail, x_hbm, idx_hbm, out_hbm, *refs):
  ibuf = refs[0]
  rows = refs[1:1 + _NB]
  gsem = refs[1 + _NB:1 + 2 * _NB]
  hsem = refs[1 + 2 * _NB:1 + 3 * _NB]
  ssem = refs[1 + 3 * _NB:1 + 4 * _NB]

  big = (nchunks + _NW - 1) // _NW
  cut = nchunks - (big - 1) * _NW

  w = lax.axis_index("s") * 2 + lax.axis_index("c")
  nj = jnp.where(w < cut, big, big - 1)
  base_chunk = jnp.where(w < cut, w * big, cut * big + (w - cut) * (big - 1))

  last_nj = big if (_NW - 1) < cut else big - 1
  if tail == _CHUNK:
    @pl.when(w < cut)
    def _sa():
      pltpu.sync_copy(idx_hbm.at[pl.ds(base_chunk * _CHUNK, big * _CHUNK)],
                      ibuf)
    if cut < _NW:
      @pl.when(w >= cut)
      def _sb():
        pltpu.sync_copy(
            idx_hbm.at[pl.ds(base_chunk * _CHUNK, (big - 1) * _CHUNK)],
            ibuf.at[pl.ds(0, (big - 1) * _CHUNK)])
  else:
    main = (last_nj - 1) * _CHUNK

    @pl.when(jnp.logical_and(w < cut, w != _NW - 1))
    def _sa():
      pltpu.sync_copy(idx_hbm.at[pl.ds(base_chunk * _CHUNK, big * _CHUNK)],
                      ibuf)
    if cut < _NW - 1:
      @pl.when(jnp.logical_and(w >= cut, w != _NW - 1))
      def _sb():
        pltpu.sync_copy(
            idx_hbm.at[pl.ds(base_chunk * _CHUNK, (big - 1) * _CHUNK)],
            ibuf.at[pl.ds(0, (big - 1) * _CHUNK)])

    @pl.when(w == _NW - 1)
    def _sc():
      pltpu.sync_copy(idx_hbm.at[pl.ds(base_chunk * _CHUNK, main)],
                      ibuf.at[pl.ds(0, main)])
      pltpu.sync_copy(idx_hbm.at[pl.ds(base_chunk * _CHUNK + main, tail)],
                      ibuf.at[pl.ds(main, tail)])
      z = jnp.zeros((16,), jnp.int32)
      for g in range((_CHUNK - tail) // 16):
        ibuf[pl.ds(main + tail + g * 16, 16)] = z

  _H = _CHUNK // 2

  def start_gather(j, p):
    pltpu.async_copy(
        x_hbm.at[ibuf.at[pl.ds(j * _CHUNK, _H)]],
        rows[p].at[pl.ds(0, _H)], gsem[p])
    pltpu.async_copy(
        x_hbm.at[ibuf.at[pl.ds(j * _CHUNK + _H, _H)]],
        rows[p].at[pl.ds(_H, _H)], hsem[p])

  def wait_gather(p):
    pltpu.make_async_copy(
        x_hbm.at[ibuf.at[pl.ds(0, _H)]],
        rows[p].at[pl.ds(0, _H)], gsem[p]).wait()
    pltpu.make_async_copy(
        x_hbm.at[ibuf.at[pl.ds(0, _H)]],
        rows[p].at[pl.ds(_H, _H)], hsem[p]).wait()

  def start_store(j, p):
    gc = base_chunk + j
    if tail == _CHUNK:
      pltpu.async_copy(rows[p], out_hbm.at[pl.ds(gc * _CHUNK, _CHUNK)], ssem[p])
    else:
      @pl.when(gc == nchunks - 1)
      def _t():
        pltpu.async_copy(rows[p].at[pl.ds(0, tail)],
                         out_hbm.at[pl.ds(gc * _CHUNK, tail)], ssem[p])

      @pl.when(gc != nchunks - 1)
      def _f():
        pltpu.async_copy(rows[p], out_hbm.at[pl.ds(gc * _CHUNK, _CHUNK)],
                         ssem[p])

  def wait_store(p, is_tail):
    n = tail if is_tail else _CHUNK
    pltpu.make_async_copy(rows[p].at[pl.ds(0, n)],
                          out_hbm.at[pl.ds(0, n)], ssem[p]).wait()

  for p in range(_NB - 1):
    @pl.when(p < nj)
    def _prime(p=p):
      start_gather(p, p)

  nrounds = (nj + _NB - 1) // _NB

  @pl.loop(0, nrounds)
  def _round(r):
    for p in range(_NB):
      j = r * _NB + p

      @pl.when(j < nj)
      def _body(j=j, p=p):
        pm1 = (p - 1) % _NB

        @pl.when(j >= 1)
        def _drain_prev():
          wait_store(pm1, False)

        @pl.when(j + _NB - 1 < nj)
        def _prefetch():
          start_gather(j + _NB - 1, pm1)

        wait_gather(p)
        start_store(j, p)

  last_p = lax.rem(nj - 1, _NB)
  last_is_tail = (base_chunk + nj - 1) == (nchunks - 1)
  for p in range(_NB):
    @pl.when(last_p == p)
    def _drain_last(p=p):
      if tail == _CHUNK:
        wait_store(p, False)
      else:
        @pl.when(last_is_tail)
        def _t():
          wait_store(p, True)

        @pl.when(jnp.logical_not(last_is_tail))
        def _f():
          wait_store(p, False)


@functools.partial(jax.jit, static_argnums=(2, 3, 4))
def _gather(x, idx_pad, n_out, nchunks, tail):
  big = (nchunks + _NW - 1) // _NW
  mesh = plsc.VectorSubcoreMesh(core_axis_name="c", subcore_axis_name="s")
  run = pl.kernel(
      functools.partial(_gather_body, nchunks, tail),
      out_type=jax.ShapeDtypeStruct((n_out, _D), jnp.float32),
      mesh=mesh,
      scratch_types=[pltpu.VMEM((big * _CHUNK,), jnp.int32)]
      + [pltpu.VMEM((_CHUNK, _D), jnp.float32) for _ in range(_NB)]
      + [pltpu.SemaphoreType.DMA for _ in range(3 * _NB)],
  )
  return run(x, idx_pad)


def kernel(x, upsample_inds):
  n_out = upsample_inds.shape[0]
  idx = upsample_inds[:, 0].astype(jnp.int32)
  nchunks = (n_out + _CHUNK - 1) // _CHUNK
  tail = n_out - (nchunks - 1) * _CHUNK
  return _gather(x, idx, n_out, nchunks, tail)
